# Initial kernel scaffold; baseline (speedup 1.0000x reference)
import functools

import jax
import jax.numpy as jnp
from jax import lax
from jax.experimental import pallas as pl
from jax.experimental.pallas import tpu as pltpu
from jax.experimental.pallas import tpu_sc as plsc

N = 10000
E = 320000
D_IN = 128
D_HID = 64
N_CLASSES = 41

NC = 2
NS = 16
LANES = 16
NW = NC * NS

CHUNK = 128
CPT = -(-E // (CHUNK * NW))
EPAD = CPT * CHUNK * NW
NPAD = 10016
RPT = NPAD // NS
D2 = 48

_mesh = plsc.VectorSubcoreMesh(core_axis_name="c", subcore_axis_name="s")


@functools.partial(
    pl.kernel,
    out_type=jax.ShapeDtypeStruct((NC, NPAD, LANES), jnp.float32),
    mesh=_mesh,
    scratch_types=[
        pltpu.VMEM((CPT, CHUNK), jnp.int32),
        pltpu.VMEM((CHUNK, LANES), jnp.float32),
        pltpu.VMEM((RPT, LANES), jnp.float32),
        pltpu.VMEM_SHARED((NPAD, LANES), jnp.float32),
    ],
)
def _deg_kernel(dst3, ones_h, zeros_h, out, dst_v, ones_v, slab_v, acc):
    c = lax.axis_index("c")
    s = lax.axis_index("s")
    wid = c * NS + s
    pltpu.sync_copy(dst3.at[wid], dst_v)
    pltpu.sync_copy(ones_h, ones_v)
    base = s * RPT
    pltpu.sync_copy(zeros_h.at[pl.ds(base, RPT)], slab_v)
    pltpu.sync_copy(slab_v, acc.at[pl.ds(base, RPT)])
    plsc.subcore_barrier()

    def body(j, carry):
        pltpu.sync_copy(ones_v, acc.at[dst_v.at[j]], add=True)
        return carry

    lax.fori_loop(0, CPT, body, 0)
    plsc.subcore_barrier()
    pltpu.sync_copy(acc.at[pl.ds(base, RPT)], slab_v)
    pltpu.sync_copy(slab_v, out.at[c, pl.ds(base, RPT)])


def _make_agg(D):
    @functools.partial(
        pl.kernel,
        out_type=jax.ShapeDtypeStruct((NC, NPAD, D), jnp.float32),
        mesh=_mesh,
        scratch_types=[
            pltpu.VMEM((CPT, CHUNK), jnp.int32),
            pltpu.VMEM((CPT, CHUNK), jnp.int32),
            pltpu.VMEM((CHUNK, D), jnp.float32),
            pltpu.VMEM((RPT, D), jnp.float32),
            pltpu.VMEM_SHARED((NPAD, D), jnp.float32),
            pltpu.SemaphoreType.DMA,
        ],
    )
    def agg(src3, dst3, h_hbm, zeros_h, out, src_v, dst_v, rows_v, slab_v, acc, gsem):
        c = lax.axis_index("c")
        s = lax.axis_index("s")
        wid = c * NS + s
        pltpu.sync_copy(src3.at[wid], src_v)
        pltpu.sync_copy(dst3.at[wid], dst_v)
        base = s * RPT
        pltpu.sync_copy(zeros_h.at[pl.ds(base, RPT)], slab_v)
        pltpu.sync_copy(slab_v, acc.at[pl.ds(base, RPT)])
        plsc.subcore_barrier()

        def body(j, carry):
            pltpu.async_copy(h_hbm.at[src_v.at[j]], rows_v, gsem).wait()
            pltpu.sync_copy(rows_v, acc.at[dst_v.at[j]], add=True)
            return carry

        lax.fori_loop(0, CPT, body, 0)
        plsc.subcore_barrier()
        pltpu.sync_copy(acc.at[pl.ds(base, RPT)], slab_v)
        pltpu.sync_copy(slab_v, out.at[c, pl.ds(base, RPT)])

    return agg


_agg64 = _make_agg(D_HID)
_agg48 = _make_agg(D2)


BR = 1000


def _dinv_of(degp_ref):
    deg = degp_ref[0, :, 0:1] + degp_ref[1, :, 0:1] + 1.0
    return lax.rsqrt(deg)


def _mm1_body(degp_ref, x_ref, w1_ref, h1p_ref):
    dinv = _dinv_of(degp_ref)
    h = jnp.dot(x_ref[...], w1_ref[...], preferred_element_type=jnp.float32)
    h1p_ref[...] = h * dinv


def _mm2_body(degp_ref, y_ref, h1p_ref, b1_ref, w2_ref, h2p_ref):
    dinv = _dinv_of(degp_ref)
    agg = dinv * (y_ref[0] + y_ref[1] + h1p_ref[...]) + b1_ref[...]
    o = jnp.maximum(agg, 0.0)
    h2p_ref[...] = jnp.dot(o, w2_ref[...], preferred_element_type=jnp.float32) * dinv


def _out_body(degp_ref, y_ref, h2p_ref, b2_ref, out_ref):
    dinv = _dinv_of(degp_ref)
    h = dinv * (y_ref[0] + y_ref[1] + h2p_ref[...]) + b2_ref[...]
    h = h[:, :N_CLASSES]
    m = jnp.max(h, axis=1, keepdims=True)
    lse = jnp.log(jnp.sum(jnp.exp(h - m), axis=1, keepdims=True)) + m
    out_ref[...] = h - lse


def _deg_spec():
    return pl.BlockSpec((NC, BR, LANES), lambda i: (0, i, 0))


_mm1 = pl.pallas_call(
    _mm1_body,
    grid=(N // BR,),
    in_specs=[
        _deg_spec(),
        pl.BlockSpec((BR, D_IN), lambda i: (i, 0)),
        pl.BlockSpec((D_IN, D_HID), lambda i: (0, 0)),
    ],
    out_specs=pl.BlockSpec((BR, D_HID), lambda i: (i, 0)),
    out_shape=jax.ShapeDtypeStruct((N, D_HID), jnp.float32),
)

_mm2 = pl.pallas_call(
    _mm2_body,
    grid=(N // BR,),
    in_specs=[
        _deg_spec(),
        pl.BlockSpec((NC, BR, D_HID), lambda i: (0, i, 0)),
        pl.BlockSpec((BR, D_HID), lambda i: (i, 0)),
        pl.BlockSpec((1, D_HID), lambda i: (0, 0)),
        pl.BlockSpec((D_HID, D2), lambda i: (0, 0)),
    ],
    out_specs=pl.BlockSpec((BR, D2), lambda i: (i, 0)),
    out_shape=jax.ShapeDtypeStruct((N, D2), jnp.float32),
)

_outk = pl.pallas_call(
    _out_body,
    grid=(N // BR,),
    in_specs=[
        _deg_spec(),
        pl.BlockSpec((NC, BR, D2), lambda i: (0, i, 0)),
        pl.BlockSpec((BR, D2), lambda i: (i, 0)),
        pl.BlockSpec((1, D2), lambda i: (0, 0)),
    ],
    out_specs=pl.BlockSpec((BR, N_CLASSES), lambda i: (i, 0)),
    out_shape=jax.ShapeDtypeStruct((N, N_CLASSES), jnp.float32),
)


@jax.jit
def kernel(x, edge_index, W1, b1, W2, b2):
    src = edge_index[0].astype(jnp.int32)
    dst = edge_index[1].astype(jnp.int32)
    padi = jnp.full((EPAD - E,), N, jnp.int32)
    src3 = jnp.concatenate([src, padi]).reshape(NW, CPT, CHUNK)
    dst3 = jnp.concatenate([dst, padi]).reshape(NW, CPT, CHUNK)

    ones_h = jnp.ones((CHUNK, LANES), jnp.float32)
    degp = _deg_kernel(dst3, ones_h, jnp.zeros((NPAD, LANES), jnp.float32))

    h1p = _mm1(degp, x, W1)
    h1p_pad = jnp.pad(h1p, ((0, NPAD - N), (0, 0)))
    y1 = _agg64(src3, dst3, h1p_pad, jnp.zeros((NPAD, D_HID), jnp.float32))

    b1r = b1.reshape(1, D_HID)
    w2p = jnp.pad(W2, ((0, 0), (0, D2 - N_CLASSES)))
    b2r = jnp.pad(b2, (0, D2 - N_CLASSES)).reshape(1, D2)
    h2p = _mm2(degp, y1, h1p, b1r, w2p)
    h2p_pad = jnp.pad(h2p, ((0, NPAD - N), (0, 0)))
    y2 = _agg48(src3, dst3, h2p_pad, jnp.zeros((NPAD, D2), jnp.float32))

    return _outk(degp, y2, h2p, b2r)


# speedup vs baseline: 19.9207x; 19.9207x over previous
import functools

import jax
import jax.numpy as jnp
from jax import lax
from jax.experimental import pallas as pl
from jax.experimental.pallas import tpu as pltpu
from jax.experimental.pallas import tpu_sc as plsc

N = 10000
E = 320000
D_IN = 128
D_HID = 64
N_CLASSES = 41

NC = 2
NS = 16
LANES = 16
NW = NC * NS

CHUNK = 128
CPT = -(-E // (CHUNK * NW))
EPAD = CPT * CHUNK * NW
NPAD = 10112
RPT = NPAD // NS
D2 = 48

_mesh = plsc.VectorSubcoreMesh(core_axis_name="c", subcore_axis_name="s")
_sc_params = pltpu.CompilerParams(use_tc_tiling_on_sc=False)


@functools.partial(
    pl.kernel,
    out_type=jax.ShapeDtypeStruct((NC, NPAD, LANES), jnp.float32),
    mesh=_mesh,
    scratch_types=[
        pltpu.VMEM((CPT, CHUNK), jnp.int32),
        pltpu.VMEM((CHUNK, LANES), jnp.float32),
        pltpu.VMEM((RPT, LANES), jnp.float32),
        pltpu.VMEM_SHARED((NPAD, LANES), jnp.float32),
    ],
    compiler_params=_sc_params,
)
def _deg_kernel(dst3, ones_h, zeros_h, out, dst_v, ones_v, slab_v, acc):
    c = lax.axis_index("c")
    s = lax.axis_index("s")
    wid = c * NS + s
    pltpu.sync_copy(dst3.at[wid], dst_v)
    pltpu.sync_copy(ones_h, ones_v)
    base = s * RPT
    pltpu.sync_copy(zeros_h.at[pl.ds(base, RPT)], slab_v)
    pltpu.sync_copy(slab_v, acc.at[pl.ds(base, RPT)])
    plsc.subcore_barrier()

    def body(j, carry):
        pltpu.sync_copy(ones_v, acc.at[dst_v.at[j]], add=True)
        return carry

    lax.fori_loop(0, CPT, body, 0)
    plsc.subcore_barrier()
    pltpu.sync_copy(acc.at[pl.ds(base, RPT)], slab_v)
    pltpu.sync_copy(slab_v, out.at[c, pl.ds(base, RPT)])


def _make_agg(D):
    @functools.partial(
        pl.kernel,
        out_type=jax.ShapeDtypeStruct((NC, NPAD, D), jnp.float32),
        mesh=_mesh,
        scratch_types=[
            pltpu.VMEM((CPT, CHUNK), jnp.int32),
            pltpu.VMEM((CPT, CHUNK), jnp.int32),
            pltpu.VMEM((CHUNK, D), jnp.float32),
            pltpu.VMEM((RPT, D), jnp.float32),
            pltpu.VMEM_SHARED((NPAD, D), jnp.float32),
            pltpu.SemaphoreType.DMA,
        ],
        compiler_params=_sc_params,
    )
    def agg(src3, dst3, h_hbm, zeros_h, out, src_v, dst_v, rows_v, slab_v, acc, gsem):
        c = lax.axis_index("c")
        s = lax.axis_index("s")
        wid = c * NS + s
        pltpu.sync_copy(src3.at[wid], src_v)
        pltpu.sync_copy(dst3.at[wid], dst_v)
        base = s * RPT
        pltpu.sync_copy(zeros_h.at[pl.ds(base, RPT)], slab_v)
        pltpu.sync_copy(slab_v, acc.at[pl.ds(base, RPT)])
        plsc.subcore_barrier()

        def body(j, carry):
            pltpu.async_copy(h_hbm.at[src_v.at[j]], rows_v, gsem).wait()
            pltpu.sync_copy(rows_v, acc.at[dst_v.at[j]], add=True)
            return carry

        lax.fori_loop(0, CPT, body, 0)
        plsc.subcore_barrier()
        pltpu.sync_copy(acc.at[pl.ds(base, RPT)], slab_v)
        pltpu.sync_copy(slab_v, out.at[c, pl.ds(base, RPT)])

    return agg


_agg64 = _make_agg(D_HID)
_agg48 = _make_agg(D2)


BR = 1000


def _dinv_of(degp_ref):
    deg = degp_ref[0, :, 0:1] + degp_ref[1, :, 0:1] + 1.0
    return lax.rsqrt(deg)


def _mm1_body(degp_ref, x_ref, w1_ref, h1p_ref):
    dinv = _dinv_of(degp_ref)
    h = jnp.dot(x_ref[...], w1_ref[...], preferred_element_type=jnp.float32)
    h1p_ref[...] = h * dinv


def _mm2_body(degp_ref, y_ref, h1p_ref, b1_ref, w2_ref, h2p_ref):
    dinv = _dinv_of(degp_ref)
    agg = dinv * (y_ref[0] + y_ref[1] + h1p_ref[...]) + b1_ref[...]
    o = jnp.maximum(agg, 0.0)
    h2p_ref[...] = jnp.dot(o, w2_ref[...], preferred_element_type=jnp.float32) * dinv


def _out_body(degp_ref, y_ref, h2p_ref, b2_ref, out_ref):
    dinv = _dinv_of(degp_ref)
    h = dinv * (y_ref[0] + y_ref[1] + h2p_ref[...]) + b2_ref[...]
    h = h[:, :N_CLASSES]
    m = jnp.max(h, axis=1, keepdims=True)
    lse = jnp.log(jnp.sum(jnp.exp(h - m), axis=1, keepdims=True)) + m
    out_ref[...] = h - lse


def _deg_spec():
    return pl.BlockSpec((NC, BR, LANES), lambda i: (0, i, 0))


_mm1 = pl.pallas_call(
    _mm1_body,
    grid=(N // BR,),
    in_specs=[
        _deg_spec(),
        pl.BlockSpec((BR, D_IN), lambda i: (i, 0)),
        pl.BlockSpec((D_IN, D_HID), lambda i: (0, 0)),
    ],
    out_specs=pl.BlockSpec((BR, D_HID), lambda i: (i, 0)),
    out_shape=jax.ShapeDtypeStruct((N, D_HID), jnp.float32),
)

_mm2 = pl.pallas_call(
    _mm2_body,
    grid=(N // BR,),
    in_specs=[
        _deg_spec(),
        pl.BlockSpec((NC, BR, D_HID), lambda i: (0, i, 0)),
        pl.BlockSpec((BR, D_HID), lambda i: (i, 0)),
        pl.BlockSpec((1, D_HID), lambda i: (0, 0)),
        pl.BlockSpec((D_HID, D2), lambda i: (0, 0)),
    ],
    out_specs=pl.BlockSpec((BR, D2), lambda i: (i, 0)),
    out_shape=jax.ShapeDtypeStruct((N, D2), jnp.float32),
)

_outk = pl.pallas_call(
    _out_body,
    grid=(N // BR,),
    in_specs=[
        _deg_spec(),
        pl.BlockSpec((NC, BR, D2), lambda i: (0, i, 0)),
        pl.BlockSpec((BR, D2), lambda i: (i, 0)),
        pl.BlockSpec((1, D2), lambda i: (0, 0)),
    ],
    out_specs=pl.BlockSpec((BR, N_CLASSES), lambda i: (i, 0)),
    out_shape=jax.ShapeDtypeStruct((N, N_CLASSES), jnp.float32),
)


@jax.jit
def kernel(x, edge_index, W1, b1, W2, b2):
    src = edge_index[0].astype(jnp.int32)
    dst = edge_index[1].astype(jnp.int32)
    padi = jnp.full((EPAD - E,), N, jnp.int32)
    src3 = jnp.concatenate([src, padi]).reshape(NW, CPT, CHUNK)
    dst3 = jnp.concatenate([dst, padi]).reshape(NW, CPT, CHUNK)

    ones_h = jnp.ones((CHUNK, LANES), jnp.float32)
    degp = _deg_kernel(dst3, ones_h, jnp.zeros((NPAD, LANES), jnp.float32))

    h1p = _mm1(degp, x, W1)
    h1p_pad = jnp.pad(h1p, ((0, NPAD - N), (0, 0)))
    y1 = _agg64(src3, dst3, h1p_pad, jnp.zeros((NPAD, D_HID), jnp.float32))

    b1r = b1.reshape(1, D_HID)
    w2p = jnp.pad(W2, ((0, 0), (0, D2 - N_CLASSES)))
    b2r = jnp.pad(b2, (0, D2 - N_CLASSES)).reshape(1, D2)
    h2p = _mm2(degp, y1, h1p, b1r, w2p)
    h2p_pad = jnp.pad(h2p, ((0, NPAD - N), (0, 0)))
    y2 = _agg48(src3, dst3, h2p_pad, jnp.zeros((NPAD, D2), jnp.float32))

    return _outk(degp, y2, h2p, b2r)
